# Initial kernel scaffold; baseline (speedup 1.0000x reference)
import jax
import jax.numpy as jnp
from jax.experimental import pallas as pl


def kernel(b_rows, b_cols, b_vals, matrix_A):
    raise NotImplementedError("write your pallas kernel here")


# speedup vs baseline: 8.7455x; 8.7455x over previous
import functools

import jax
import jax.numpy as jnp
from jax import lax
from jax.experimental import pallas as pl
from jax.experimental.pallas import tpu as pltpu
from jax.experimental.pallas import tpu_sc as plsc

_NC = 2
_NS = 16
_NW = _NC * _NS
_B = 128
_L = 16


def _bcast_lane(v, l):
    idx = jnp.full((_L, 1), l, dtype=jnp.int32)
    return lax.gather(
        v, idx,
        lax.GatherDimensionNumbers(
            offset_dims=(), collapsed_slice_dims=(0,), start_index_map=(0,)),
        (1,),
        mode=lax.GatherScatterMode.PROMISE_IN_BOUNDS)


def _sc_scatter_mm(cols2d, vals2d, rows2d, matrix_a, zeros_nd, G, N, D):
    mesh = plsc.VectorSubcoreMesh(core_axis_name="c", subcore_axis_name="s")
    rows_per_tile = N // _NS

    @functools.partial(
        pl.kernel,
        mesh=mesh,
        out_type=jax.ShapeDtypeStruct((_NC, N, D), jnp.float32),
        scratch_types=[
            pltpu.VMEM((G, _B), jnp.int32),
            pltpu.VMEM((G, _B), jnp.float32),
            pltpu.VMEM((G, _B), jnp.int32),
            pltpu.VMEM((_B, D), jnp.float32),
            pltpu.VMEM_SHARED((N, D), jnp.float32),
            pltpu.SemaphoreType.DMA,
        ],
        compiler_params=pltpu.CompilerParams(use_tc_tiling_on_sc=False),
    )
    def k(cols_hbm, vals_hbm, rows_hbm, a_hbm, z_hbm, out_hbm,
          cols_v, vals_v, rows_v, gbuf, acc, sem):
        c = lax.axis_index("c")
        s = lax.axis_index("s")
        wid = s * _NC + c

        pltpu.sync_copy(z_hbm.at[pl.ds(s * rows_per_tile, rows_per_tile)],
                        acc.at[pl.ds(s * rows_per_tile, rows_per_tile)])
        pltpu.sync_copy(cols_hbm.at[wid], cols_v)
        pltpu.sync_copy(vals_hbm.at[wid], vals_v)
        pltpu.sync_copy(rows_hbm.at[wid], rows_v)
        plsc.subcore_barrier()

        def group(g, carry):
            pltpu.async_copy(a_hbm.at[cols_v.at[g]], gbuf, sem).wait()
            for j in range(_B // _L):
                v16 = vals_v[g, pl.ds(j * _L, _L)]
                for l in range(_L):
                    bv = _bcast_lane(v16, l)
                    r = j * _L + l
                    for d in range(D // _L):
                        sl = pl.ds(d * _L, _L)
                        gbuf[r, sl] = gbuf[r, sl] * bv
            pltpu.sync_copy(gbuf, acc.at[rows_v.at[g]], add=True)
            return carry

        lax.fori_loop(0, G, group, 0)
        plsc.subcore_barrier()
        pltpu.sync_copy(acc.at[pl.ds(s * rows_per_tile, rows_per_tile)],
                        out_hbm.at[c, pl.ds(s * rows_per_tile, rows_per_tile)])

    return k(cols2d, vals2d, rows2d, matrix_a, zeros_nd)


def _combine(partials, N, D):
    blk = 1024

    def add_body(a_ref, b_ref, o_ref):
        o_ref[...] = a_ref[...] + b_ref[...]

    return pl.pallas_call(
        add_body,
        grid=(N // blk,),
        in_specs=[pl.BlockSpec((blk, D), lambda i: (i, 0)),
                  pl.BlockSpec((blk, D), lambda i: (i, 0))],
        out_specs=pl.BlockSpec((blk, D), lambda i: (i, 0)),
        out_shape=jax.ShapeDtypeStruct((N, D), jnp.float32),
    )(partials[0], partials[1])


def kernel(b_rows, b_cols, b_vals, matrix_A):
    nnz = b_rows.shape[0]
    N, D = matrix_A.shape
    per = _NW * _B
    G = -(-nnz // per)
    pad = G * per - nnz
    cols = jnp.concatenate(
        [b_cols.astype(jnp.int32), jnp.zeros((pad,), jnp.int32)]
    ).reshape(_NW, G, _B)
    vals = jnp.concatenate(
        [b_vals, jnp.zeros((pad,), jnp.float32)]).reshape(_NW, G, _B)
    rows = jnp.concatenate(
        [b_rows.astype(jnp.int32), jnp.zeros((pad,), jnp.int32)]
    ).reshape(_NW, G, _B)
    zeros_nd = jnp.zeros((N, D), jnp.float32)
    partials = _sc_scatter_mm(cols, vals, rows, matrix_A, zeros_nd, G, N, D)
    return _combine(partials, N, D)
